# baseline (speedup 1.0000x reference)
import jax
import jax.numpy as jnp
from jax.experimental import pallas as pl
from jax.experimental.pallas import tpu as pltpu

FEATURES = 1024
BLOCK_TOKENS = 2048
BLOCK_F = 512


def _mask_fill_body(idx_ref, z_ref, zout_ref, mask_ref):
    j = pl.program_id(1)
    idxv = idx_ref[0, 0, :]
    col = jax.lax.broadcasted_iota(jnp.int32, (BLOCK_TOKENS, BLOCK_F), 1)
    m = (col + j * BLOCK_F) >= idxv[:, None]
    zout_ref[...] = jnp.where(m, jnp.float32(0.0), z_ref[...])
    mask_ref[...] = m.astype(jnp.int8)


def kernel(z, masks, idx):
    del masks
    B, S, F = z.shape
    n_tok = B * S
    n_blocks = n_tok // BLOCK_TOKENS
    n_fblocks = F // BLOCK_F
    z2 = z.reshape(n_tok, F)
    idx3 = idx.reshape(n_blocks, 1, BLOCK_TOKENS)

    zout, mask = pl.pallas_call(
        _mask_fill_body,
        grid=(n_blocks, n_fblocks),
        in_specs=[
            pl.BlockSpec((1, 1, BLOCK_TOKENS), lambda i, j: (i, 0, 0)),
            pl.BlockSpec((BLOCK_TOKENS, BLOCK_F), lambda i, j: (i, j)),
        ],
        out_specs=[
            pl.BlockSpec((BLOCK_TOKENS, BLOCK_F), lambda i, j: (i, j)),
            pl.BlockSpec((BLOCK_TOKENS, BLOCK_F), lambda i, j: (i, j)),
        ],
        out_shape=[
            jax.ShapeDtypeStruct((n_tok, F), z.dtype),
            jax.ShapeDtypeStruct((n_tok, F), jnp.int8),
        ],
        compiler_params=pltpu.CompilerParams(
            dimension_semantics=("parallel", "parallel"),
        ),
    )(idx3, z2)

    return zout.reshape(B, S, F), mask.view(jnp.bool_).reshape(B, S, F)
